# baseline (speedup 1.0000x reference)
import functools

import jax
import jax.numpy as jnp
from jax import lax
from jax.experimental import pallas as pl
from jax.experimental.pallas import tpu as pltpu
from jax.experimental.pallas import tpu_sc as plsc

_NUM_CORES = 2
_NUM_SUBCORES = 16
_NUM_WORKERS = _NUM_CORES * _NUM_SUBCORES


def kernel(x, table):
    (batch,) = x.shape
    _, embed_dim = table.shape
    b_per_w = batch // _NUM_WORKERS
    mesh = plsc.VectorSubcoreMesh(core_axis_name="c", subcore_axis_name="s")

    @functools.partial(
        pl.kernel,
        mesh=mesh,
        out_type=jax.ShapeDtypeStruct((batch, embed_dim), table.dtype),
        scratch_types=[
            pltpu.VMEM((b_per_w,), jnp.int32),
            pltpu.VMEM((b_per_w, embed_dim), table.dtype),
            pltpu.SemaphoreType.DMA,
        ],
        compiler_params=pltpu.CompilerParams(use_tc_tiling_on_sc=False),
    )
    def emb(x_hbm, table_hbm, out_hbm, idx_v, rows_v, sem):
        wid = lax.axis_index("s") * _NUM_CORES + lax.axis_index("c")
        base = wid * b_per_w
        pltpu.sync_copy(x_hbm.at[pl.ds(base, b_per_w)], idx_v)
        pltpu.async_copy(table_hbm.at[idx_v], rows_v, sem).wait()
        pltpu.sync_copy(rows_v, out_hbm.at[pl.ds(base, b_per_w)])

    return emb(x.astype(jnp.int32), table)


# speedup vs baseline: 3.8732x; 3.8732x over previous
import functools

import jax
import jax.numpy as jnp
from jax import lax
from jax.experimental import pallas as pl
from jax.experimental.pallas import tpu as pltpu
from jax.experimental.pallas import tpu_sc as plsc

_NUM_CORES = 2
_NUM_SUBCORES = 16
_NUM_WORKERS = _NUM_CORES * _NUM_SUBCORES
_LANES = 16
_CHUNK = 16


def kernel(x, table):
    (batch,) = x.shape
    n_rows, embed_dim = table.shape
    table_t = table.T
    b_per_w = batch // _NUM_WORKERS
    mesh = plsc.VectorSubcoreMesh(core_axis_name="c", subcore_axis_name="s")

    @functools.partial(
        pl.kernel,
        mesh=mesh,
        out_type=jax.ShapeDtypeStruct((embed_dim, batch), table.dtype),
        scratch_types=[
            pltpu.VMEM((b_per_w,), jnp.int32),
            pltpu.VMEM((_CHUNK, embed_dim, 128), jnp.float32),
            pltpu.VMEM((embed_dim, b_per_w), jnp.float32),
            pltpu.SemaphoreType.DMA,
            pltpu.SemaphoreType.DMA,
        ],
        compiler_params=pltpu.CompilerParams(needs_layout_passes=False),
    )
    def emb(x_hbm, table_hbm, out_hbm, idx_v, stage_v, outb_v, sem_in, sem_out):
        wid = lax.axis_index("s") * _NUM_CORES + lax.axis_index("c")
        base = wid * b_per_w
        pltpu.sync_copy(x_hbm.at[pl.ds(base, b_per_w)], idx_v)

        def chunk_body(ci, _):
            cbase = ci * _CHUNK

            ivec = idx_v[pl.ds(cbase, _CHUNK)]
            for k in range(_CHUNK):
                col = pl.multiple_of((ivec[k] // 128) * 128, 128)
                pltpu.async_copy(
                    table_hbm.at[:, pl.ds(col, 128)], stage_v.at[k], sem_in
                )

            def drain(k, _):
                pltpu.make_async_copy(
                    table_hbm.at[:, pl.ds(0, 128)], stage_v.at[k], sem_in
                ).wait()
                return ()

            lax.fori_loop(0, _CHUNK, drain, (), unroll=8)

            def extract(g, _):
                idx16 = idx_v[pl.ds(cbase + g * _LANES, _LANES)]
                lane = lax.rem(idx16, 128)
                kvec = lax.iota(jnp.int32, _LANES) + g * _LANES

                def comp(c, _):
                    cvec = jnp.full((_LANES,), c, dtype=jnp.int32)
                    v = plsc.load_gather(stage_v, [kvec, cvec, lane])
                    outb_v[c, pl.ds(cbase + g * _LANES, _LANES)] = v
                    return ()

                lax.fori_loop(0, embed_dim, comp, (), unroll=8)
                return ()

            lax.fori_loop(0, _CHUNK // _LANES, extract, (), unroll=1)
            return ()

        lax.fori_loop(0, b_per_w // _CHUNK, chunk_body, (), unroll=1)
        pltpu.sync_copy(outb_v, out_hbm.at[:, pl.ds(base, b_per_w)])

    out_t = emb(x.astype(jnp.int32), table_t)
    return out_t.T
